# baseline (device time: 25312 ns/iter reference)
import jax
import jax.numpy as jnp
from jax import lax
from jax.experimental import pallas as pl
from jax.experimental.pallas import tpu as pltpu

N_DEV = 4
BLOCK_M = 256


def kernel(x, dy, gamma):
    del gamma
    m, d = x.shape
    n_blocks = m // BLOCK_M

    def body(x_ref, dy_ref, out_ref, acc_ref, comm_ref, send_sems, recv_sems):
        i = pl.program_id(0)

        @pl.when(i == 0)
        def _():
            acc_ref[...] = jnp.zeros_like(acc_ref)

        xb = x_ref[...]
        dyb = dy_ref[...]
        mu = jnp.mean(xb, axis=1, keepdims=True)
        xc = xb - mu
        var = jnp.mean(xc * xc, axis=1, keepdims=True)
        rstd = lax.rsqrt(var + 1e-5)
        xhat = xc * rstd
        dgamma = jnp.sum(dyb * xhat, axis=0, keepdims=True)
        dbeta = jnp.sum(dyb, axis=0, keepdims=True)
        acc_ref[...] += jnp.concatenate([dgamma, dbeta], axis=0)

        @pl.when(i == n_blocks - 1)
        def _():
            my_pos = lax.axis_index("i")
            left = (my_pos - 1) % N_DEV
            right = (my_pos + 1) % N_DEV

            barrier_sem = pltpu.get_barrier_semaphore()
            for nbr in (left, right):
                pl.semaphore_signal(
                    barrier_sem, inc=1,
                    device_id=(nbr,), device_id_type=pl.DeviceIdType.MESH,
                )
            pl.semaphore_wait(barrier_sem, 2)

            comm_ref[0] = acc_ref[...]
            out_ref[...] = acc_ref[...]
            for h in range(N_DEV - 1):
                rdma = pltpu.make_async_remote_copy(
                    src_ref=comm_ref.at[h],
                    dst_ref=comm_ref.at[h + 1],
                    send_sem=send_sems.at[h],
                    recv_sem=recv_sems.at[h],
                    device_id=(right,),
                    device_id_type=pl.DeviceIdType.MESH,
                )
                rdma.start()
                rdma.wait()
                out_ref[...] += comm_ref[h + 1]

    return pl.pallas_call(
        body,
        grid=(n_blocks,),
        in_specs=[
            pl.BlockSpec((BLOCK_M, d), lambda i: (i, 0)),
            pl.BlockSpec((BLOCK_M, d), lambda i: (i, 0)),
        ],
        out_specs=pl.BlockSpec((2, d), lambda i: (0, 0)),
        out_shape=jax.ShapeDtypeStruct((2, d), jnp.float32),
        scratch_shapes=[
            pltpu.VMEM((2, d), jnp.float32),
            pltpu.VMEM((N_DEV, 2, d), jnp.float32),
            pltpu.SemaphoreType.DMA((N_DEV - 1,)),
            pltpu.SemaphoreType.DMA((N_DEV - 1,)),
        ],
        compiler_params=pltpu.CompilerParams(collective_id=0),
    )(x, dy)


# device time: 21989 ns/iter; 1.1511x vs baseline; 1.1511x over previous
import jax
import jax.numpy as jnp
from jax import lax
from jax.experimental import pallas as pl
from jax.experimental.pallas import tpu as pltpu

N_DEV = 4
BLOCK_M = 256


def kernel(x, dy, gamma):
    del gamma
    m, d = x.shape
    n_blocks = m // BLOCK_M

    def body(x_ref, dy_ref, out_ref, acc_ref, comm_ref, send_sems, recv_sems):
        i = pl.program_id(0)

        @pl.when(i == 0)
        def _():
            acc_ref[...] = jnp.zeros_like(acc_ref)

        xb = x_ref[...]
        dyb = dy_ref[...]
        mu = jnp.mean(xb, axis=1, keepdims=True)
        xc = xb - mu
        var = jnp.mean(xc * xc, axis=1, keepdims=True)
        rstd = lax.rsqrt(var + 1e-5)
        xhat = xc * rstd
        dgamma = jnp.sum(dyb * xhat, axis=0, keepdims=True)
        dbeta = jnp.sum(dyb, axis=0, keepdims=True)
        acc_ref[...] += jnp.concatenate([dgamma, dbeta], axis=0)

        @pl.when(i == n_blocks - 1)
        def _():
            my_pos = lax.axis_index("i")

            barrier_sem = pltpu.get_barrier_semaphore()
            for k in range(1, N_DEV):
                pl.semaphore_signal(
                    barrier_sem, inc=1,
                    device_id=((my_pos + k) % N_DEV,),
                    device_id_type=pl.DeviceIdType.MESH,
                )
            pl.semaphore_wait(barrier_sem, N_DEV - 1)

            comm_ref[my_pos] = acc_ref[...]
            sends = []
            for k in range(1, N_DEV):
                peer = (my_pos + k) % N_DEV
                send = pltpu.make_async_remote_copy(
                    src_ref=comm_ref.at[my_pos],
                    dst_ref=comm_ref.at[my_pos],
                    send_sem=send_sems.at[k - 1],
                    recv_sem=recv_sems.at[my_pos],
                    device_id=(peer,),
                    device_id_type=pl.DeviceIdType.MESH,
                )
                send.start()
                sends.append(send)

            out_ref[...] = acc_ref[...]
            for k in range(1, N_DEV):
                src = (my_pos + k) % N_DEV
                recv = pltpu.make_async_remote_copy(
                    src_ref=comm_ref.at[src],
                    dst_ref=comm_ref.at[src],
                    send_sem=send_sems.at[k - 1],
                    recv_sem=recv_sems.at[src],
                    device_id=(my_pos,),
                    device_id_type=pl.DeviceIdType.MESH,
                )
                recv.wait_recv()
                out_ref[...] += comm_ref[src]
            for send in sends:
                send.wait_send()

    return pl.pallas_call(
        body,
        grid=(n_blocks,),
        in_specs=[
            pl.BlockSpec((BLOCK_M, d), lambda i: (i, 0)),
            pl.BlockSpec((BLOCK_M, d), lambda i: (i, 0)),
        ],
        out_specs=pl.BlockSpec((2, d), lambda i: (0, 0)),
        out_shape=jax.ShapeDtypeStruct((2, d), jnp.float32),
        scratch_shapes=[
            pltpu.VMEM((2, d), jnp.float32),
            pltpu.VMEM((N_DEV, 2, d), jnp.float32),
            pltpu.SemaphoreType.DMA((N_DEV - 1,)),
            pltpu.SemaphoreType.DMA((N_DEV,)),
        ],
        compiler_params=pltpu.CompilerParams(collective_id=0),
    )(x, dy)


# device time: 14651 ns/iter; 1.7277x vs baseline; 1.5009x over previous
import os

import jax
import jax.numpy as jnp
from jax import lax
from jax.experimental import pallas as pl
from jax.experimental.pallas import tpu as pltpu

N_DEV = 4
BLOCK_M = 256

_MODE = os.environ.get("KMODE", "full")


def kernel(x, dy, gamma):
    del gamma
    m, d = x.shape
    n_blocks = m // BLOCK_M

    def body(x_ref, dy_ref, out_ref, acc_ref, comm_ref, send_sems, recv_sems):
        i = pl.program_id(0)

        @pl.when(i == 0)
        def _():
            acc_ref[...] = jnp.zeros_like(acc_ref)

        xb = x_ref[...]
        dyb = dy_ref[...]
        if _MODE == "dma_only":
            acc_ref[...] += jnp.concatenate(
                [jnp.sum(xb, axis=0, keepdims=True),
                 jnp.sum(dyb, axis=0, keepdims=True)], axis=0)
        else:
            mu = jnp.mean(xb, axis=1, keepdims=True)
            xc = xb - mu
            var = jnp.mean(xc * xc, axis=1, keepdims=True)
            rstd = lax.rsqrt(var + 1e-5)
            xhat = xc * rstd
            dgamma = jnp.sum(dyb * xhat, axis=0, keepdims=True)
            dbeta = jnp.sum(dyb, axis=0, keepdims=True)
            acc_ref[...] += jnp.concatenate([dgamma, dbeta], axis=0)

        @pl.when((i == n_blocks - 1) & (_MODE == "full"))
        def _():
            my_pos = lax.axis_index("i")

            barrier_sem = pltpu.get_barrier_semaphore()
            for k in range(1, N_DEV):
                pl.semaphore_signal(
                    barrier_sem, inc=1,
                    device_id=((my_pos + k) % N_DEV,),
                    device_id_type=pl.DeviceIdType.MESH,
                )
            pl.semaphore_wait(barrier_sem, N_DEV - 1)

            comm_ref[my_pos] = acc_ref[...]
            sends = []
            for k in range(1, N_DEV):
                peer = (my_pos + k) % N_DEV
                send = pltpu.make_async_remote_copy(
                    src_ref=comm_ref.at[my_pos],
                    dst_ref=comm_ref.at[my_pos],
                    send_sem=send_sems.at[k - 1],
                    recv_sem=recv_sems.at[my_pos],
                    device_id=(peer,),
                    device_id_type=pl.DeviceIdType.MESH,
                )
                send.start()
                sends.append(send)

            out_ref[...] = acc_ref[...]
            for k in range(1, N_DEV):
                src = (my_pos + k) % N_DEV
                recv = pltpu.make_async_remote_copy(
                    src_ref=comm_ref.at[src],
                    dst_ref=comm_ref.at[src],
                    send_sem=send_sems.at[k - 1],
                    recv_sem=recv_sems.at[src],
                    device_id=(my_pos,),
                    device_id_type=pl.DeviceIdType.MESH,
                )
                recv.wait_recv()
                out_ref[...] += comm_ref[src]
            for send in sends:
                send.wait_send()

    return pl.pallas_call(
        body,
        grid=(n_blocks,),
        in_specs=[
            pl.BlockSpec((BLOCK_M, d), lambda i: (i, 0)),
            pl.BlockSpec((BLOCK_M, d), lambda i: (i, 0)),
        ],
        out_specs=pl.BlockSpec((2, d), lambda i: (0, 0)),
        out_shape=jax.ShapeDtypeStruct((2, d), jnp.float32),
        scratch_shapes=[
            pltpu.VMEM((2, d), jnp.float32),
            pltpu.VMEM((N_DEV, 2, d), jnp.float32),
            pltpu.SemaphoreType.DMA((N_DEV - 1,)),
            pltpu.SemaphoreType.DMA((N_DEV,)),
        ],
        compiler_params=pltpu.CompilerParams(collective_id=0),
    )(x, dy)
